# baseline (speedup 1.0000x reference)
import dataclasses
import functools

import jax
import jax.numpy as jnp
from jax import lax
from jax.experimental import pallas as pl
from jax.experimental.pallas import tpu as pltpu
from jax.experimental.pallas import tpu_sc as plsc

S = 16384
M = 768
E = 64
TOPK = 2
CAP = 512
TBLK = 256
NBLK = S // TBLK
PAD_ROWS = S + TBLK


def _router_tc_kernel(x_ref, wg_ref, tri_ref, xpad_ref, inv1_ref, inv2_ref,
                      c1i_ref, loss_ref, inv1_s, inv2_s, c1_ref, me_ref,
                      c2_scr):
    i = pl.program_id(0)

    @pl.when(i == 0)
    def _init():
        inv1_s[...] = jnp.zeros((E, CAP), jnp.float32)
        inv2_s[...] = jnp.zeros((E, CAP), jnp.float32)
        c1_ref[...] = jnp.zeros((1, E), jnp.float32)
        me_ref[...] = jnp.zeros((1, E), jnp.float32)
        c2_scr[...] = jnp.zeros((1, E), jnp.float32)

    @pl.when(i < NBLK)
    def _body():
        x = x_ref[...]
        xpad_ref[...] = x
        logits = lax.dot_general(
            x, wg_ref[...], (((1,), (1,)), ((), ())),
            preferred_element_type=jnp.float32)

        mx = jnp.max(logits, axis=1, keepdims=True)
        ex = jnp.exp(logits - mx)
        gates = ex / jnp.sum(ex, axis=1, keepdims=True)
        me_ref[...] += jnp.sum(gates, axis=0, keepdims=True)

        eids = lax.broadcasted_iota(jnp.int32, (TBLK, E), 1)
        top1 = jnp.argmax(logits, axis=1)
        m1 = (eids == top1[:, None]).astype(jnp.float32)
        masked = jnp.where(m1 > 0, -jnp.inf, logits)
        top2 = jnp.argmax(masked, axis=1)
        m2 = (eids == top2[:, None]).astype(jnp.float32)

        tri = tri_ref[...]
        cum1 = lax.dot_general(tri, m1.astype(jnp.bfloat16),
                               (((1,), (0,)), ((), ())),
                               preferred_element_type=jnp.float32)
        cum2 = lax.dot_general(tri, m2.astype(jnp.bfloat16),
                               (((1,), (0,)), ((), ())),
                               preferred_element_type=jnp.float32)

        rank1 = (jnp.sum(cum1 * m1, axis=1) - 1.0
                 + jnp.sum(c1_ref[...] * m1, axis=1)).astype(jnp.int32)
        rank2 = (jnp.sum(cum2 * m2, axis=1) - 1.0
                 + jnp.sum(c2_scr[...] * m2, axis=1)).astype(jnp.int32)

        cap_iota = lax.broadcasted_iota(jnp.int32, (TBLK, CAP), 1)
        q1 = (cap_iota == rank1[:, None]).astype(jnp.bfloat16)
        q2 = (cap_iota == rank2[:, None]).astype(jnp.bfloat16)
        tokp1 = (i * TBLK + 1
                 + lax.broadcasted_iota(jnp.int32, (TBLK, 1), 0))
        hi = (tokp1 // 128).astype(jnp.float32)
        lo = (tokp1 % 128).astype(jnp.float32)
        dn = (((0,), (0,)), ((), ()))
        a1h = (m1 * hi).astype(jnp.bfloat16)
        a1l = (m1 * lo).astype(jnp.bfloat16)
        a2h = (m2 * hi).astype(jnp.bfloat16)
        a2l = (m2 * lo).astype(jnp.bfloat16)
        s1h = lax.dot_general(a1h, q1, dn,
                              preferred_element_type=jnp.float32)
        s1l = lax.dot_general(a1l, q1, dn,
                              preferred_element_type=jnp.float32)
        s2h = lax.dot_general(a2h, q2, dn,
                              preferred_element_type=jnp.float32)
        s2l = lax.dot_general(a2l, q2, dn,
                              preferred_element_type=jnp.float32)
        inv1_s[...] += s1h * 128.0 + s1l
        inv2_s[...] += s2h * 128.0 + s2l

        c1_ref[...] += jnp.sum(m1, axis=0, keepdims=True)
        c2_scr[...] += jnp.sum(m2, axis=0, keepdims=True)

    @pl.when(i == NBLK)
    def _pad():
        xpad_ref[...] = jnp.zeros((TBLK, M), jnp.float32)
        inv1_ref[...] = inv1_s[...].astype(jnp.int32)
        inv2_ref[...] = inv2_s[...].astype(jnp.int32)
        c1i_ref[...] = c1_ref[...].astype(jnp.int32)
        loss_ref[...] = jnp.sum(me_ref[...] * c1_ref[...],
                                keepdims=True)[:, :1] * (E / (S * S))


def _run_router(x, wg):
    return pl.pallas_call(
        _router_tc_kernel,
        grid=(NBLK + 1,),
        in_specs=[
            pl.BlockSpec((TBLK, M), lambda i: (jnp.minimum(i, NBLK - 1), 0)),
            pl.BlockSpec((E, M), lambda i: (0, 0)),
            pl.BlockSpec((TBLK, TBLK), lambda i: (0, 0)),
        ],
        out_specs=[
            pl.BlockSpec((TBLK, M), lambda i: (i, 0)),
            pl.BlockSpec((E, CAP), lambda i: (0, 0)),
            pl.BlockSpec((E, CAP), lambda i: (0, 0)),
            pl.BlockSpec((1, E), lambda i: (0, 0)),
            pl.BlockSpec((1, 1), lambda i: (0, 0)),
        ],
        out_shape=[
            jax.ShapeDtypeStruct((PAD_ROWS, M), jnp.float32),
            jax.ShapeDtypeStruct((E, CAP), jnp.int32),
            jax.ShapeDtypeStruct((E, CAP), jnp.int32),
            jax.ShapeDtypeStruct((1, E), jnp.int32),
            jax.ShapeDtypeStruct((1, 1), jnp.float32),
        ],
        scratch_shapes=[
            pltpu.VMEM((E, CAP), jnp.float32),
            pltpu.VMEM((E, CAP), jnp.float32),
            pltpu.VMEM((1, E), jnp.float32),
            pltpu.VMEM((1, E), jnp.float32),
            pltpu.VMEM((1, E), jnp.float32),
        ],
    )(x, wg, _tri_const())


@functools.lru_cache(maxsize=1)
def _tri_const():
    import numpy as np
    return jnp.asarray(np.tril(np.ones((TBLK, TBLK), np.float32)),
                       dtype=jnp.bfloat16)


NW = 32
PER_W = E * CAP // NW
CH = 64
NCH = PER_W // CH
GCH = 128


def _sc_gather_kernel(xpad_hbm, inv1_hbm, inv2_hbm, c1_hbm, out_hbm,
                      idx_v, inv1_v, inv2_v, c1_v, buf0, buf1,
                      gsem0, gsem1, osem0, osem1):
    wid = lax.axis_index("s") * 2 + lax.axis_index("c")
    base = wid * PER_W
    e0 = wid * 2

    pltpu.sync_copy(inv1_hbm.at[pl.ds(e0 * CAP, 2 * CAP)], inv1_v)
    pltpu.sync_copy(inv2_hbm.at[pl.ds(e0 * CAP, 2 * CAP)], inv2_v)
    chunk0 = (e0 // 16) * 16
    pltpu.sync_copy(c1_hbm.at[pl.ds(chunk0, 16)], c1_v)

    lane = lax.iota(jnp.int32, 16)
    for el in range(2):
        c1b = plsc.load_gather(c1_v, [jnp.full((16,), e0 % 16 + el,
                                                jnp.int32)])
        for g in range(CAP // 16):
            c_vec = lane + 16 * g
            use2 = c_vec >= c1b
            pos2 = jnp.clip(c_vec - c1b, 0, CAP - 1) + el * CAP
            v1 = inv1_v[pl.ds(el * CAP + 16 * g, 16)]
            v2 = plsc.load_gather(inv2_v, [pos2])
            tok = jnp.where(use2, v2, v1)
            idx_v[pl.ds(el * CAP + 16 * g, 16)] = jnp.where(
                tok > 0, tok - 1, S)
    bufs = (buf0, buf1)
    gsems = (gsem0, gsem1)
    osems = (osem0, osem1)

    def fire_gather(c):
        return pltpu.async_copy(
            xpad_hbm.at[idx_v.at[pl.ds(c * CH, CH)]], bufs[c % 2],
            gsems[c % 2])

    def fire_out(c):
        return pltpu.async_copy(
            bufs[c % 2], out_hbm.at[pl.ds(base + c * CH, CH)], osems[c % 2])

    g = {0: fire_gather(0)}
    o = {}
    for c in range(NCH):
        if c >= 1:
            o[c - 1].wait()
        if c + 1 < NCH:
            g[c + 1] = fire_gather(c + 1)
        g[c].wait()
        o[c] = fire_out(c)
    o[NCH - 1].wait()


def _sc_compiler_params():
    cp = pltpu.CompilerParams()
    if "needs_layout_passes" in pltpu.CompilerParams.__dataclass_fields__:
        cp = dataclasses.replace(cp, needs_layout_passes=False)
    return cp


@functools.lru_cache(maxsize=1)
def _sc_gather():
    return pl.kernel(
        _sc_gather_kernel,
        out_type=jax.ShapeDtypeStruct((E * CAP, M), jnp.float32),
        mesh=plsc.VectorSubcoreMesh(core_axis_name="c", subcore_axis_name="s"),
        compiler_params=_sc_compiler_params(),
        scratch_types=[
            pltpu.VMEM((PER_W,), jnp.int32),
            pltpu.VMEM((2 * CAP,), jnp.int32),
            pltpu.VMEM((2 * CAP,), jnp.int32),
            pltpu.VMEM((16,), jnp.int32),
            pltpu.VMEM((CH, M), jnp.float32),
            pltpu.VMEM((CH, M), jnp.float32),
            pltpu.SemaphoreType.DMA,
            pltpu.SemaphoreType.DMA,
            pltpu.SemaphoreType.DMA,
            pltpu.SemaphoreType.DMA,
        ],
    )


def kernel(input, wg_weight):
    xpad, inv1, inv2, c1, loss = _run_router(input, wg_weight)
    dispatched = _sc_gather()(xpad, inv1.reshape(-1), inv2.reshape(-1),
                              c1.reshape(E)).reshape(E, CAP, M)
    return dispatched, loss.reshape(())


# speedup vs baseline: 1.1044x; 1.1044x over previous
import dataclasses
import functools

import jax
import jax.numpy as jnp
from jax import lax
from jax.experimental import pallas as pl
from jax.experimental.pallas import tpu as pltpu
from jax.experimental.pallas import tpu_sc as plsc

S = 16384
M = 768
E = 64
TOPK = 2
CAP = 512
TBLK = 256
NBLK = S // TBLK
PAD_ROWS = S + TBLK


def _router_tc_kernel(x_ref, wg_ref, tri_ref, xpad_ref, inv1_ref, inv2_ref,
                      c1_ref, me_ref, c2_scr):
    i = pl.program_id(0)

    @pl.when(i == 0)
    def _init():
        inv1_ref[...] = jnp.zeros((E, CAP), jnp.float32)
        inv2_ref[...] = jnp.zeros((E, CAP), jnp.float32)
        c1_ref[...] = jnp.zeros((1, E), jnp.float32)
        me_ref[...] = jnp.zeros((1, E), jnp.float32)
        c2_scr[...] = jnp.zeros((1, E), jnp.float32)

    @pl.when(i < NBLK)
    def _body():
        x = x_ref[...]
        xpad_ref[...] = x
        logits = lax.dot_general(
            x, wg_ref[...], (((1,), (1,)), ((), ())),
            preferred_element_type=jnp.float32)

        mx = jnp.max(logits, axis=1, keepdims=True)
        ex = jnp.exp(logits - mx)
        gates = ex / jnp.sum(ex, axis=1, keepdims=True)
        me_ref[...] += jnp.sum(gates, axis=0, keepdims=True)

        eids = lax.broadcasted_iota(jnp.int32, (TBLK, E), 1)
        top1 = jnp.argmax(logits, axis=1)
        m1 = (eids == top1[:, None]).astype(jnp.float32)
        masked = jnp.where(m1 > 0, -jnp.inf, logits)
        top2 = jnp.argmax(masked, axis=1)
        m2 = (eids == top2[:, None]).astype(jnp.float32)

        tri = tri_ref[...]
        cum1 = lax.dot_general(tri, m1.astype(jnp.bfloat16),
                               (((1,), (0,)), ((), ())),
                               preferred_element_type=jnp.float32)
        cum2 = lax.dot_general(tri, m2.astype(jnp.bfloat16),
                               (((1,), (0,)), ((), ())),
                               preferred_element_type=jnp.float32)

        rank1 = (jnp.sum(cum1 * m1, axis=1) - 1.0
                 + jnp.sum(c1_ref[...] * m1, axis=1)).astype(jnp.int32)
        rank2 = (jnp.sum(cum2 * m2, axis=1) - 1.0
                 + jnp.sum(c2_scr[...] * m2, axis=1)).astype(jnp.int32)

        cap_iota = lax.broadcasted_iota(jnp.int32, (TBLK, CAP), 1)
        q1 = (cap_iota == rank1[:, None]).astype(jnp.bfloat16)
        q2 = (cap_iota == rank2[:, None]).astype(jnp.bfloat16)
        tokp1 = (i * TBLK + 1
                 + lax.broadcasted_iota(jnp.int32, (TBLK, 1), 0))
        hi = (tokp1 // 128).astype(jnp.float32)
        lo = (tokp1 % 128).astype(jnp.float32)
        dn = (((0,), (0,)), ((), ()))
        a1h = (m1 * hi).astype(jnp.bfloat16)
        a1l = (m1 * lo).astype(jnp.bfloat16)
        a2h = (m2 * hi).astype(jnp.bfloat16)
        a2l = (m2 * lo).astype(jnp.bfloat16)
        s1h = lax.dot_general(a1h, q1, dn,
                              preferred_element_type=jnp.float32)
        s1l = lax.dot_general(a1l, q1, dn,
                              preferred_element_type=jnp.float32)
        s2h = lax.dot_general(a2h, q2, dn,
                              preferred_element_type=jnp.float32)
        s2l = lax.dot_general(a2l, q2, dn,
                              preferred_element_type=jnp.float32)
        inv1_ref[...] += s1h * 128.0 + s1l
        inv2_ref[...] += s2h * 128.0 + s2l

        c1_ref[...] += jnp.sum(m1, axis=0, keepdims=True)
        c2_scr[...] += jnp.sum(m2, axis=0, keepdims=True)

    @pl.when(i == NBLK)
    def _pad():
        xpad_ref[...] = jnp.zeros((TBLK, M), jnp.float32)


def _run_router(x, wg):
    return pl.pallas_call(
        _router_tc_kernel,
        grid=(NBLK + 1,),
        in_specs=[
            pl.BlockSpec((TBLK, M), lambda i: (jnp.minimum(i, NBLK - 1), 0)),
            pl.BlockSpec((E, M), lambda i: (0, 0)),
            pl.BlockSpec((TBLK, TBLK), lambda i: (0, 0)),
        ],
        out_specs=[
            pl.BlockSpec((TBLK, M), lambda i: (i, 0)),
            pl.BlockSpec((E, CAP), lambda i: (0, 0)),
            pl.BlockSpec((E, CAP), lambda i: (0, 0)),
            pl.BlockSpec((1, E), lambda i: (0, 0)),
            pl.BlockSpec((1, E), lambda i: (0, 0)),
        ],
        out_shape=[
            jax.ShapeDtypeStruct((PAD_ROWS, M), jnp.float32),
            jax.ShapeDtypeStruct((E, CAP), jnp.float32),
            jax.ShapeDtypeStruct((E, CAP), jnp.float32),
            jax.ShapeDtypeStruct((1, E), jnp.float32),
            jax.ShapeDtypeStruct((1, E), jnp.float32),
        ],
        scratch_shapes=[pltpu.VMEM((1, E), jnp.float32)],
    )(x, wg, _tri_const())


@functools.lru_cache(maxsize=1)
def _tri_const():
    import numpy as np
    return jnp.asarray(np.tril(np.ones((TBLK, TBLK), np.float32)),
                       dtype=jnp.bfloat16)


NW = 32
PER_W = E * CAP // NW
CH = 64
NCH = PER_W // CH
GCH = 128


def _sc_gather_kernel(xpad_hbm, inv1_hbm, inv2_hbm, c1_hbm, out_hbm,
                      idx_v, inv1_v, inv2_v, c1_v, buf0, buf1,
                      gsem0, gsem1, osem0, osem1):
    wid = lax.axis_index("s") * 2 + lax.axis_index("c")
    base = wid * PER_W
    e0 = wid * 2

    pltpu.sync_copy(inv1_hbm.at[pl.ds(e0 * CAP, 2 * CAP)], inv1_v)
    pltpu.sync_copy(inv2_hbm.at[pl.ds(e0 * CAP, 2 * CAP)], inv2_v)
    chunk0 = (e0 // 16) * 16
    pltpu.sync_copy(c1_hbm.at[pl.ds(chunk0, 16)], c1_v)

    lane = lax.iota(jnp.int32, 16)
    for el in range(2):
        c1b = plsc.load_gather(c1_v, [jnp.full((16,), e0 % 16 + el,
                                                jnp.int32)])
        for g in range(CAP // 16):
            c_vec = lane + 16 * g
            use2 = c_vec >= c1b
            pos2 = jnp.clip(c_vec - c1b, 0, CAP - 1) + el * CAP
            v1 = inv1_v[pl.ds(el * CAP + 16 * g, 16)]
            v2 = plsc.load_gather(inv2_v, [pos2])
            tok = jnp.where(use2, v2, v1)
            idx_v[pl.ds(el * CAP + 16 * g, 16)] = jnp.where(
                tok > 0, tok - 1, S)
    bufs = (buf0, buf1)
    gsems = (gsem0, gsem1)
    osems = (osem0, osem1)

    def fire_gather(c):
        return pltpu.async_copy(
            xpad_hbm.at[idx_v.at[pl.ds(c * CH, CH)]], bufs[c % 2],
            gsems[c % 2])

    def fire_out(c):
        return pltpu.async_copy(
            bufs[c % 2], out_hbm.at[pl.ds(base + c * CH, CH)], osems[c % 2])

    g = {0: fire_gather(0)}
    o = {}
    for c in range(NCH):
        if c >= 1:
            o[c - 1].wait()
        if c + 1 < NCH:
            g[c + 1] = fire_gather(c + 1)
        g[c].wait()
        o[c] = fire_out(c)
    o[NCH - 1].wait()


def _sc_compiler_params():
    cp = pltpu.CompilerParams()
    if "needs_layout_passes" in pltpu.CompilerParams.__dataclass_fields__:
        cp = dataclasses.replace(cp, needs_layout_passes=False)
    return cp


@functools.lru_cache(maxsize=1)
def _sc_gather():
    return pl.kernel(
        _sc_gather_kernel,
        out_type=jax.ShapeDtypeStruct((E * CAP, M), jnp.float32),
        mesh=plsc.VectorSubcoreMesh(core_axis_name="c", subcore_axis_name="s"),
        compiler_params=_sc_compiler_params(),
        scratch_types=[
            pltpu.VMEM((PER_W,), jnp.int32),
            pltpu.VMEM((2 * CAP,), jnp.int32),
            pltpu.VMEM((2 * CAP,), jnp.int32),
            pltpu.VMEM((16,), jnp.int32),
            pltpu.VMEM((CH, M), jnp.float32),
            pltpu.VMEM((CH, M), jnp.float32),
            pltpu.SemaphoreType.DMA,
            pltpu.SemaphoreType.DMA,
            pltpu.SemaphoreType.DMA,
            pltpu.SemaphoreType.DMA,
        ],
    )


def kernel(input, wg_weight):
    xpad, inv1f, inv2f, c1f, me = _run_router(input, wg_weight)

    inv1 = inv1f.astype(jnp.int32)
    inv2 = inv2f.astype(jnp.int32)
    c1 = c1f.astype(jnp.int32).reshape(E)

    dispatched = _sc_gather()(xpad, inv1.reshape(-1), inv2.reshape(-1),
                              c1).reshape(E, CAP, M)

    l_loss = jnp.sum(me[0] * c1f[0]) * (E / (S * S))
    return dispatched, l_loss
